# baseline (speedup 1.0000x reference)
import functools

import jax
import jax.numpy as jnp
from jax.experimental import pallas as pl
from jax.experimental.pallas import tpu as pltpu

_TILE_N = 2048
_NBUF = 4
_INV_T = 20.0


def _mm_manual(x_ref, mem_hbm, out_hbm, in_buf, out_buf, in_sem, out_sem,
               *, n_chunks):
    def in_copy(i, slot):
        return pltpu.make_async_copy(
            mem_hbm.at[pl.ds(i * _TILE_N, _TILE_N), :],
            in_buf.at[slot], in_sem.at[slot])

    def out_copy(i, slot):
        return pltpu.make_async_copy(
            out_buf.at[slot],
            out_hbm.at[:, pl.ds(i * _TILE_N, _TILE_N)], out_sem.at[slot])

    for j in range(_NBUF):
        in_copy(j, j).start()
    xb = (x_ref[...] * _INV_T).astype(jnp.bfloat16)

    def body(i, _):
        slot = jax.lax.rem(i, _NBUF)
        in_copy(i, slot).wait()

        @pl.when(i >= _NBUF)
        def _():
            out_copy(i - _NBUF, slot).wait()

        m = in_buf[slot].astype(jnp.bfloat16)
        out_buf[slot] = jax.lax.dot_general(
            xb, m, (((1,), (1,)), ((), ())),
            preferred_element_type=jnp.float32)
        out_copy(i, slot).start()

        @pl.when(i + _NBUF < n_chunks)
        def _():
            in_copy(i + _NBUF, slot).start()

        return 0

    jax.lax.fori_loop(0, n_chunks, body, 0)
    for j in range(n_chunks - _NBUF, n_chunks):
        out_copy(j, j % _NBUF).wait()


def kernel(x, mem_feat):
    q, k = x.shape
    n = mem_feat.shape[0]
    n_chunks = n // _TILE_N
    return pl.pallas_call(
        functools.partial(_mm_manual, n_chunks=n_chunks),
        in_specs=[
            pl.BlockSpec((q, k), lambda: (0, 0)),
            pl.BlockSpec(memory_space=pl.MemorySpace.ANY),
        ],
        out_specs=pl.BlockSpec(memory_space=pl.MemorySpace.ANY),
        out_shape=jax.ShapeDtypeStruct((q, n), jnp.float32),
        scratch_shapes=[
            pltpu.VMEM((_NBUF, _TILE_N, k), jnp.float32),
            pltpu.VMEM((_NBUF, q, _TILE_N), jnp.float32),
            pltpu.SemaphoreType.DMA((_NBUF,)),
            pltpu.SemaphoreType.DMA((_NBUF,)),
        ],
    )(x, mem_feat)


# speedup vs baseline: 1.0148x; 1.0148x over previous
import jax
import jax.numpy as jnp
from jax.experimental import pallas as pl
from jax.experimental.pallas import tpu as pltpu

_TILE_N = 4096
_INV_T = 20.0


def _mm_kernel(x_ref, m_ref, o_ref):
    x = (x_ref[...] * _INV_T).astype(jnp.bfloat16)
    m = m_ref[...].astype(jnp.bfloat16)
    o_ref[...] = jax.lax.dot_general(
        x, m, (((1,), (1,)), ((), ())),
        preferred_element_type=jnp.float32)


def kernel(x, mem_feat):
    q, k = x.shape
    n = mem_feat.shape[0]
    return pl.pallas_call(
        _mm_kernel,
        grid=(n // _TILE_N,),
        in_specs=[
            pl.BlockSpec((q, k), lambda i: (0, 0)),
            pl.BlockSpec((_TILE_N, k), lambda i: (i, 0)),
        ],
        out_specs=pl.BlockSpec((q, _TILE_N), lambda i: (0, i)),
        out_shape=jax.ShapeDtypeStruct((q, n), jnp.float32),
    )(x, mem_feat)


# speedup vs baseline: 1.0156x; 1.0007x over previous
import jax
import jax.numpy as jnp
from jax.experimental import pallas as pl

_TILE_N = 4096
_INV_T = 20.0


def _mm_kernel(x_ref, m_ref, o_ref):
    x = (x_ref[...] * _INV_T).astype(jnp.bfloat16)
    m = m_ref[...].astype(jnp.bfloat16)
    o_ref[...] = jax.lax.dot_general(
        x, m, (((1,), (1,)), ((), ())),
        preferred_element_type=jnp.float32)


def kernel(x, mem_feat):
    q, k = x.shape
    n = mem_feat.shape[0]
    return pl.pallas_call(
        _mm_kernel,
        grid=(n // _TILE_N,),
        in_specs=[
            pl.BlockSpec((q, k), lambda i: (0, 0)),
            pl.BlockSpec((_TILE_N, k), lambda i: (i, 0)),
        ],
        out_specs=pl.BlockSpec((q, _TILE_N), lambda i: (0, i)),
        out_shape=jax.ShapeDtypeStruct((q, n), jnp.float32),
    )(x, mem_feat)
